# baseline (speedup 1.0000x reference)
import functools

import jax
import jax.numpy as jnp
from jax import lax
from jax.experimental import pallas as pl
from jax.experimental.pallas import tpu as pltpu
from jax.experimental.pallas import tpu_sc as plsc

VOCAB = 1000000
D = 64
C = 200
B = 4096
N = B * C
NC, NS, L = 2, 16, 16
NW = NC * NS
PER_W = N // NW
CHUNK = 128
NCH = PER_W // CHUNK
UNROLL = 4

_mesh = plsc.VectorSubcoreMesh(core_axis_name="c", subcore_axis_name="s")


@functools.partial(
    pl.kernel,
    out_type=jax.ShapeDtypeStruct((N, D), jnp.float32),
    mesh=_mesh,
    scratch_types=[
        pltpu.VMEM((NCH, CHUNK), jnp.int32),
        pltpu.VMEM((2, CHUNK, D), jnp.float32),
        pltpu.VMEM((2, CHUNK, D), jnp.float32),
        pltpu.SemaphoreType.DMA,
        pltpu.SemaphoreType.DMA,
        pltpu.SemaphoreType.DMA,
        pltpu.SemaphoreType.DMA,
    ],
    compiler_params=pltpu.CompilerParams(use_tc_tiling_on_sc=False),
)
def _embed(idx_hbm, table_hbm, out_hbm, idx_v, rin_v, rout_v, g0, g1, o0, o1):
    wid = lax.axis_index("s") * NC + lax.axis_index("c")
    base = wid * PER_W
    pltpu.sync_copy(idx_hbm.at[wid], idx_v)

    gsems = (g0, g1)
    osems = (o0, o1)

    iota = lax.convert_element_type(
        lax.broadcasted_iota(jnp.int32, (L,), 0), jnp.float32)
    iks = [(iota + (k * L + 1)) * (1.0 / D) for k in range(4)]

    def gather(j, b):
        pltpu.async_copy(table_hbm.at[idx_v.at[j]], rin_v.at[b], gsems[b])

    def wait(sem, b):
        pltpu.make_async_copy(out_hbm.at[pl.ds(0, CHUNK)], rin_v.at[b], sem).wait()

    gather(0, 0)
    gather(1, 1)

    def group_body(g, _):
        for b in range(2):
            j = 2 * g + b
            wait(gsems[b], b)
            @pl.when(g >= 1)
            def _():
                wait(osems[b], b)

            s = lax.rem(j * CHUNK, C)

            def row_body(i0, _):
                for u in range(UNROLL):
                    i = i0 * UNROLL + u
                    r = lax.rem(s + i, C)
                    p = lax.convert_element_type(r + 1, jnp.float32) * (1.0 / C)
                    c1 = 1.0 - p
                    c2 = 1.0 - 2.0 * p
                    for k in range(4):
                        sl = pl.ds(k * L, L)
                        rout_v[b, i, sl] = rin_v[b, i, sl] + (c1 - iks[k] * c2)
                return 0

            lax.fori_loop(0, CHUNK // UNROLL, row_body, 0, unroll=1)

            @pl.when(g < NCH // 2 - 1)
            def _():
                gather(j + 2, b)

            pltpu.async_copy(
                rout_v.at[b], out_hbm.at[pl.ds(base + j * CHUNK, CHUNK)], osems[b])
        return 0

    lax.fori_loop(0, NCH // 2, group_body, 0)

    wait(osems[0], 0)
    wait(osems[1], 1)


def kernel(inputs, table):
    idx = inputs.reshape(NW, NCH, CHUNK).astype(jnp.int32)
    out = _embed(idx, table)
    return out.reshape(B, C, D)
